# baseline (speedup 1.0000x reference)
import functools

import jax
import jax.numpy as jnp
from jax import lax
from jax.experimental import pallas as pl
from jax.experimental.pallas import tpu as pltpu
from jax.experimental.pallas import tpu_sc as plsc

E = 8
C = 768
H = 3072
T = 2048
TILE = 128
R = T * 2 + E * TILE
NT = R // TILE
NH = 2
HC = H // NH
CHUNK = 256
NW = 32
TPW = T // NW


def _router_body(x_ref, wr_ref, br_ref, r1_ref, r2_ref, p1_ref, p2_ref,
                 eid_ref, valid_ref, mask_ref, cum_ref):
    logits = jnp.dot(x_ref[...], wr_ref[...],
                     preferred_element_type=jnp.float32) + br_ref[...]
    iota8 = lax.broadcasted_iota(jnp.int32, (T, E), 1)
    m1 = jnp.max(logits, axis=1, keepdims=True)
    e1 = jnp.argmax(logits, axis=1)[:, None]
    masked = jnp.where(iota8 == e1, -jnp.inf, logits)
    m2 = jnp.max(masked, axis=1, keepdims=True)
    e2 = jnp.argmax(masked, axis=1)[:, None]
    ed = jnp.exp(m2 - m1)
    p1_ref[...] = jnp.broadcast_to(1.0 / (1.0 + ed), (T, 16))
    p2_ref[...] = jnp.broadcast_to(ed / (1.0 + ed), (T, 16))
    sel1 = iota8 == e1
    sel2 = iota8 == e2
    mask_ref[...] = sel1.astype(jnp.float32) + sel2.astype(jnp.float32)

    trilc = (lax.broadcasted_iota(jnp.int32, (CHUNK, CHUNK), 0)
             >= lax.broadcasted_iota(jnp.int32, (CHUNK, CHUNK), 1)
             ).astype(jnp.float32)

    def step(i, carry):
        off = pl.multiple_of(i * CHUNK, CHUNK)
        blk = mask_ref[pl.ds(off, CHUNK), :]
        cum_blk = jnp.dot(trilc, blk, preferred_element_type=jnp.float32) + carry
        cum_ref[pl.ds(off, CHUNK), :] = cum_blk
        return cum_blk[CHUNK - 1:CHUNK, :]

    counts = lax.fori_loop(0, T // CHUNK, step, jnp.zeros((1, E), jnp.float32))

    aligned = jnp.ceil(counts / TILE) * TILE
    stri = (lax.broadcasted_iota(jnp.int32, (E, E), 0)
            < lax.broadcasted_iota(jnp.int32, (E, E), 1)).astype(jnp.float32)
    offs = jnp.dot(aligned, stri, preferred_element_type=jnp.float32)

    cum = cum_ref[...]
    r1_ref[...] = jnp.sum(jnp.where(sel1, cum - 1.0 + offs, 0.0), axis=1,
                          keepdims=True).astype(jnp.int32)
    r2_ref[...] = jnp.sum(jnp.where(sel2, cum - 1.0 + offs, 0.0), axis=1,
                          keepdims=True).astype(jnp.int32)

    starts = (lax.broadcasted_iota(jnp.int32, (NT, E), 0) * TILE
              ).astype(jnp.float32)
    offs_b = jnp.broadcast_to(offs, (NT, E))
    eid_ref[...] = jnp.sum((offs_b <= starts).astype(jnp.int32), axis=1,
                           keepdims=True) - 1
    used_b = jnp.broadcast_to(offs + counts, (NT, E))
    vmax = jnp.max(jnp.where(offs_b <= starts, used_b, 0.0), axis=1,
                   keepdims=True)
    valid_ref[...] = (starts[:, 0:1] < vmax).astype(jnp.int32)


_router = pl.pallas_call(
    _router_body,
    out_shape=[
        jax.ShapeDtypeStruct((T, 1), jnp.int32),
        jax.ShapeDtypeStruct((T, 1), jnp.int32),
        jax.ShapeDtypeStruct((T, 16), jnp.float32),
        jax.ShapeDtypeStruct((T, 16), jnp.float32),
        jax.ShapeDtypeStruct((NT, 1), jnp.int32),
        jax.ShapeDtypeStruct((NT, 1), jnp.int32),
    ],
    scratch_shapes=[
        pltpu.VMEM((T, E), jnp.float32),
        pltpu.VMEM((T, E), jnp.float32),
    ],
)


def _mlp_body(eid_ref, valid_ref, x_ref, wfc_ref, bfc_ref, wg_ref, bg_ref,
              wp_ref, bp_ref, o_ref):
    del eid_ref
    h = pl.program_id(0)
    i = pl.program_id(1)
    rows = pl.ds(pl.multiple_of(i * TILE, TILE), TILE)
    @pl.when(valid_ref[i] != 0)
    def _():
        x = x_ref[...]
        hh = jnp.dot(x, wfc_ref[0],
                     preferred_element_type=jnp.float32) + bfc_ref[0]
        g = jnp.dot(x, wg_ref[0],
                    preferred_element_type=jnp.float32) + bg_ref[0]
        og = hh * (g / (1.0 + jnp.exp(-g)))
        contrib = jnp.dot(og, wp_ref[0], preferred_element_type=jnp.float32)

        @pl.when(h == 0)
        def _():
            o_ref[rows, :] = contrib + bp_ref[0]

        @pl.when(h != 0)
        def _():
            o_ref[rows, :] = o_ref[rows, :] + contrib


_mlp = pl.pallas_call(
    _mlp_body,
    grid_spec=pltpu.PrefetchScalarGridSpec(
        num_scalar_prefetch=2,
        grid=(NH, NT),
        in_specs=[
            pl.BlockSpec((TILE, C), lambda h, i, eid, valid: (i, 0)),
            pl.BlockSpec((1, C, HC), lambda h, i, eid, valid: (eid[i], 0, h)),
            pl.BlockSpec((1, 1, HC), lambda h, i, eid, valid: (eid[i], 0, h)),
            pl.BlockSpec((1, C, HC), lambda h, i, eid, valid: (eid[i], 0, h)),
            pl.BlockSpec((1, 1, HC), lambda h, i, eid, valid: (eid[i], 0, h)),
            pl.BlockSpec((1, HC, C), lambda h, i, eid, valid: (eid[i], h, 0)),
            pl.BlockSpec((1, 1, C), lambda h, i, eid, valid: (eid[i], 0, 0)),
        ],
        out_specs=pl.BlockSpec((R, C), lambda h, i, eid, valid: (0, 0)),
    ),
    out_shape=jax.ShapeDtypeStruct((R, C), jnp.float32),
)


@functools.lru_cache(maxsize=None)
def _sc_kernels():
    mesh = plsc.VectorSubcoreMesh(core_axis_name="c", subcore_axis_name="s")

    @functools.partial(
        pl.kernel,
        mesh=mesh,
        out_type=jax.ShapeDtypeStruct((R, C), jnp.float32),
        scratch_types=[
            pltpu.VMEM((TPW, C), jnp.float32),
            pltpu.VMEM((TPW,), jnp.int32),
            pltpu.VMEM((TPW,), jnp.int32),
            pltpu.SemaphoreType.DMA,
        ],
    )
    def dispatch(xf_hbm, r1_hbm, r2_hbm, xs_hbm, xv, i1v, i2v, sem):
        wid = lax.axis_index("s") * 2 + lax.axis_index("c")
        base = wid * TPW
        pltpu.sync_copy(xf_hbm.at[pl.ds(base, TPW)], xv)
        pltpu.sync_copy(r1_hbm.at[pl.ds(base, TPW)], i1v)
        pltpu.sync_copy(r2_hbm.at[pl.ds(base, TPW)], i2v)
        pltpu.async_copy(xv, xs_hbm.at[i1v], sem).wait()
        pltpu.async_copy(xv, xs_hbm.at[i2v], sem).wait()

    @functools.partial(
        pl.kernel,
        mesh=mesh,
        out_type=jax.ShapeDtypeStruct((T, C), jnp.float32),
        scratch_types=[
            pltpu.VMEM((TPW, C), jnp.float32),
            pltpu.VMEM((TPW, C), jnp.float32),
            pltpu.VMEM((TPW,), jnp.int32),
            pltpu.VMEM((TPW, 16), jnp.float32),
            pltpu.VMEM((TPW, 16), jnp.float32),
            pltpu.SemaphoreType.DMA,
        ],
    )
    def combine(o_hbm, r1_hbm, r2_hbm, p1_hbm, p2_hbm, y_hbm, b1, b2, iv,
                pv1, pv2, sem):
        wid = lax.axis_index("s") * 2 + lax.axis_index("c")
        base = wid * TPW
        pltpu.sync_copy(r1_hbm.at[pl.ds(base, TPW)], iv)
        pltpu.async_copy(o_hbm.at[iv], b1, sem).wait()
        pltpu.sync_copy(r2_hbm.at[pl.ds(base, TPW)], iv)
        pltpu.async_copy(o_hbm.at[iv], b2, sem).wait()
        pltpu.sync_copy(p1_hbm.at[pl.ds(base, TPW)], pv1)
        pltpu.sync_copy(p2_hbm.at[pl.ds(base, TPW)], pv2)

        def row(j, carry):
            a1 = pv1[j]
            a2 = pv2[j]
            for v in range(C // 16):
                s = v * 16
                b1[j, pl.ds(s, 16)] = (a1 * b1[j, pl.ds(s, 16)]
                                       + a2 * b2[j, pl.ds(s, 16)])
            return carry

        lax.fori_loop(0, TPW, row, 0)
        pltpu.sync_copy(b1, y_hbm.at[pl.ds(base, TPW)])

    return dispatch, combine


@jax.jit
def kernel(x, w_router, b_router, w_c_fc, b_c_fc, w_gate, b_gate, w_c_proj,
           b_c_proj):
    xf = x.reshape(T, C)
    r1, r2, p1, p2, eid, valid = _router(xf, w_router, b_router.reshape(1, E))
    r1f = r1.reshape(T)
    r2f = r2.reshape(T)
    dispatch, combine = _sc_kernels()
    xs = dispatch(xf, r1f, r2f)
    o = _mlp(eid.reshape(NT), valid.reshape(NT), xs, w_c_fc, b_c_fc, w_gate,
             b_gate, w_c_proj, b_c_proj)
    return combine(o, r1f, r2f, p1, p2)


# speedup vs baseline: 1.1164x; 1.1164x over previous
import functools

import jax
import jax.numpy as jnp
from jax import lax
from jax.experimental import pallas as pl
from jax.experimental.pallas import tpu as pltpu
from jax.experimental.pallas import tpu_sc as plsc

E = 8
C = 768
H = 3072
T = 2048
TILE = 256
R = T * 2 + E * TILE
NT = R // TILE
NH = 2
HC = H // NH
CHUNK = 256
NW = 32
TPW = T // NW


def _router_body(x_ref, wr_ref, br_ref, r1_ref, r2_ref, p1_ref, p2_ref,
                 eid_ref, valid_ref, mask_ref, cum_ref):
    logits = jnp.dot(x_ref[...], wr_ref[...],
                     preferred_element_type=jnp.float32) + br_ref[...]
    iota8 = lax.broadcasted_iota(jnp.int32, (T, E), 1)
    m1 = jnp.max(logits, axis=1, keepdims=True)
    e1 = jnp.argmax(logits, axis=1)[:, None]
    masked = jnp.where(iota8 == e1, -jnp.inf, logits)
    m2 = jnp.max(masked, axis=1, keepdims=True)
    e2 = jnp.argmax(masked, axis=1)[:, None]
    ed = jnp.exp(m2 - m1)
    p1_ref[...] = jnp.broadcast_to(1.0 / (1.0 + ed), (T, 16))
    p2_ref[...] = jnp.broadcast_to(ed / (1.0 + ed), (T, 16))
    sel1 = iota8 == e1
    sel2 = iota8 == e2
    mask_ref[...] = sel1.astype(jnp.float32) + sel2.astype(jnp.float32)

    trilc = (lax.broadcasted_iota(jnp.int32, (CHUNK, CHUNK), 0)
             >= lax.broadcasted_iota(jnp.int32, (CHUNK, CHUNK), 1)
             ).astype(jnp.float32)

    def step(i, carry):
        off = pl.multiple_of(i * CHUNK, CHUNK)
        blk = mask_ref[pl.ds(off, CHUNK), :]
        cum_blk = jnp.dot(trilc, blk, preferred_element_type=jnp.float32) + carry
        cum_ref[pl.ds(off, CHUNK), :] = cum_blk
        return cum_blk[CHUNK - 1:CHUNK, :]

    counts = lax.fori_loop(0, T // CHUNK, step, jnp.zeros((1, E), jnp.float32))

    aligned = jnp.ceil(counts / TILE) * TILE
    stri = (lax.broadcasted_iota(jnp.int32, (E, E), 0)
            < lax.broadcasted_iota(jnp.int32, (E, E), 1)).astype(jnp.float32)
    offs = jnp.dot(aligned, stri, preferred_element_type=jnp.float32)

    cum = cum_ref[...]
    r1_ref[...] = jnp.sum(jnp.where(sel1, cum - 1.0 + offs, 0.0), axis=1,
                          keepdims=True).astype(jnp.int32)
    r2_ref[...] = jnp.sum(jnp.where(sel2, cum - 1.0 + offs, 0.0), axis=1,
                          keepdims=True).astype(jnp.int32)

    starts = (lax.broadcasted_iota(jnp.int32, (NT, E), 0) * TILE
              ).astype(jnp.float32)
    offs_b = jnp.broadcast_to(offs, (NT, E))
    eid_ref[...] = jnp.sum((offs_b <= starts).astype(jnp.int32), axis=1,
                           keepdims=True) - 1
    used_b = jnp.broadcast_to(offs + counts, (NT, E))
    vmax = jnp.max(jnp.where(offs_b <= starts, used_b, 0.0), axis=1,
                   keepdims=True)
    valid_ref[...] = (starts[:, 0:1] < vmax).astype(jnp.int32)


_router = pl.pallas_call(
    _router_body,
    out_shape=[
        jax.ShapeDtypeStruct((T, 1), jnp.int32),
        jax.ShapeDtypeStruct((T, 1), jnp.int32),
        jax.ShapeDtypeStruct((T, 16), jnp.float32),
        jax.ShapeDtypeStruct((T, 16), jnp.float32),
        jax.ShapeDtypeStruct((NT, 1), jnp.int32),
        jax.ShapeDtypeStruct((NT, 1), jnp.int32),
    ],
    scratch_shapes=[
        pltpu.VMEM((T, E), jnp.float32),
        pltpu.VMEM((T, E), jnp.float32),
    ],
)


def _mlp_body(eid_ref, valid_ref, x_ref, wfc_ref, bfc_ref, wg_ref, bg_ref,
              wp_ref, bp_ref, o_ref):
    del eid_ref
    h = pl.program_id(0)
    i = pl.program_id(1)
    rows = pl.ds(pl.multiple_of(i * TILE, TILE), TILE)
    @pl.when(valid_ref[i] != 0)
    def _():
        x = x_ref[...]
        hh = jnp.dot(x, wfc_ref[0],
                     preferred_element_type=jnp.float32) + bfc_ref[0]
        g = jnp.dot(x, wg_ref[0],
                    preferred_element_type=jnp.float32) + bg_ref[0]
        og = hh * (g / (1.0 + jnp.exp(-g)))
        contrib = jnp.dot(og, wp_ref[0], preferred_element_type=jnp.float32)

        @pl.when(h == 0)
        def _():
            o_ref[rows, :] = contrib + bp_ref[0]

        @pl.when(h != 0)
        def _():
            o_ref[rows, :] = o_ref[rows, :] + contrib


_mlp = pl.pallas_call(
    _mlp_body,
    grid_spec=pltpu.PrefetchScalarGridSpec(
        num_scalar_prefetch=2,
        grid=(NH, NT),
        in_specs=[
            pl.BlockSpec((TILE, C), lambda h, i, eid, valid: (i, 0)),
            pl.BlockSpec((1, C, HC), lambda h, i, eid, valid: (eid[i], 0, h)),
            pl.BlockSpec((1, 1, HC), lambda h, i, eid, valid: (eid[i], 0, h)),
            pl.BlockSpec((1, C, HC), lambda h, i, eid, valid: (eid[i], 0, h)),
            pl.BlockSpec((1, 1, HC), lambda h, i, eid, valid: (eid[i], 0, h)),
            pl.BlockSpec((1, HC, C), lambda h, i, eid, valid: (eid[i], h, 0)),
            pl.BlockSpec((1, 1, C), lambda h, i, eid, valid: (eid[i], 0, 0)),
        ],
        out_specs=pl.BlockSpec((R, C), lambda h, i, eid, valid: (0, 0)),
    ),
    out_shape=jax.ShapeDtypeStruct((R, C), jnp.float32),
)


@functools.lru_cache(maxsize=None)
def _sc_kernels():
    mesh = plsc.VectorSubcoreMesh(core_axis_name="c", subcore_axis_name="s")

    @functools.partial(
        pl.kernel,
        mesh=mesh,
        out_type=jax.ShapeDtypeStruct((R, C), jnp.float32),
        scratch_types=[
            pltpu.VMEM((TPW, C), jnp.float32),
            pltpu.VMEM((TPW,), jnp.int32),
            pltpu.VMEM((TPW,), jnp.int32),
            pltpu.SemaphoreType.DMA,
        ],
    )
    def dispatch(xf_hbm, r1_hbm, r2_hbm, xs_hbm, xv, i1v, i2v, sem):
        wid = lax.axis_index("s") * 2 + lax.axis_index("c")
        base = wid * TPW
        pltpu.sync_copy(xf_hbm.at[pl.ds(base, TPW)], xv)
        pltpu.sync_copy(r1_hbm.at[pl.ds(base, TPW)], i1v)
        pltpu.sync_copy(r2_hbm.at[pl.ds(base, TPW)], i2v)
        pltpu.async_copy(xv, xs_hbm.at[i1v], sem).wait()
        pltpu.async_copy(xv, xs_hbm.at[i2v], sem).wait()

    @functools.partial(
        pl.kernel,
        mesh=mesh,
        out_type=jax.ShapeDtypeStruct((T, C), jnp.float32),
        scratch_types=[
            pltpu.VMEM((TPW, C), jnp.float32),
            pltpu.VMEM((TPW, C), jnp.float32),
            pltpu.VMEM((TPW,), jnp.int32),
            pltpu.VMEM((TPW, 16), jnp.float32),
            pltpu.VMEM((TPW, 16), jnp.float32),
            pltpu.SemaphoreType.DMA,
        ],
    )
    def combine(o_hbm, r1_hbm, r2_hbm, p1_hbm, p2_hbm, y_hbm, b1, b2, iv,
                pv1, pv2, sem):
        wid = lax.axis_index("s") * 2 + lax.axis_index("c")
        base = wid * TPW
        pltpu.sync_copy(r1_hbm.at[pl.ds(base, TPW)], iv)
        pltpu.async_copy(o_hbm.at[iv], b1, sem).wait()
        pltpu.sync_copy(r2_hbm.at[pl.ds(base, TPW)], iv)
        pltpu.async_copy(o_hbm.at[iv], b2, sem).wait()
        pltpu.sync_copy(p1_hbm.at[pl.ds(base, TPW)], pv1)
        pltpu.sync_copy(p2_hbm.at[pl.ds(base, TPW)], pv2)

        def row(j, carry):
            a1 = pv1[j]
            a2 = pv2[j]
            for v in range(C // 16):
                s = v * 16
                b1[j, pl.ds(s, 16)] = (a1 * b1[j, pl.ds(s, 16)]
                                       + a2 * b2[j, pl.ds(s, 16)])
            return carry

        lax.fori_loop(0, TPW, row, 0)
        pltpu.sync_copy(b1, y_hbm.at[pl.ds(base, TPW)])

    return dispatch, combine


@jax.jit
def kernel(x, w_router, b_router, w_c_fc, b_c_fc, w_gate, b_gate, w_c_proj,
           b_c_proj):
    xf = x.reshape(T, C)
    r1, r2, p1, p2, eid, valid = _router(xf, w_router, b_router.reshape(1, E))
    r1f = r1.reshape(T)
    r2f = r2.reshape(T)
    dispatch, combine = _sc_kernels()
    xs = dispatch(xf, r1f, r2f)
    o = _mlp(eid.reshape(NT), valid.reshape(NT), xs, w_c_fc, b_c_fc, w_gate,
             b_gate, w_c_proj, b_c_proj)
    return combine(o, r1f, r2f, p1, p2)
